# Initial kernel scaffold; baseline (speedup 1.0000x reference)
import jax
import jax.numpy as jnp
from jax.experimental import pallas as pl


def kernel(x, edge_index, W1, b1, W2, b2, W3, b3, eps):
    raise NotImplementedError("write your pallas kernel here")


# speedup vs baseline: 154.9061x; 154.9061x over previous
import functools

import jax
import jax.numpy as jnp
from jax import lax
from jax.experimental import pallas as pl
from jax.experimental.pallas import tpu as pltpu
from jax.experimental.pallas import tpu_sc as plsc

N_NODES = 100000
N_EDGES = 6400000
LANES = 128
N_PAD = 100352
PER_TILE = N_PAD // 16
ROWS = N_EDGES // LANES
CHUNK_ROWS = 16
N_CHUNKS = ROWS // CHUNK_ROWS
N_WORKERS = 32
MAX_CHUNKS_PER_TILE = (N_CHUNKS + N_WORKERS - 1) // N_WORKERS


def _sc_scatter(x_pad, src2d, dst2d, zeros_pad):
    mesh = plsc.VectorSubcoreMesh(core_axis_name="c", subcore_axis_name="s")

    @functools.partial(
        pl.kernel,
        out_type=[
            jax.ShapeDtypeStruct((N_PAD,), jnp.float32),
            jax.ShapeDtypeStruct((N_PAD,), jnp.float32),
        ],
        mesh=mesh,
        compiler_params=pltpu.CompilerParams(needs_layout_passes=False),
        scratch_types=[
            pltpu.VMEM((N_PAD,), jnp.float32),
            pltpu.VMEM((CHUNK_ROWS, LANES), jnp.int32),
            pltpu.VMEM((CHUNK_ROWS, LANES), jnp.int32),
            pltpu.VMEM((CHUNK_ROWS, LANES), jnp.float32),
            pltpu.VMEM((PER_TILE,), jnp.float32),
            pltpu.VMEM_SHARED((N_PAD,), jnp.float32),
        ],
    )
    def sc_kernel(x_hbm, src_hbm, dst_hbm, z_hbm, p0_hbm, p1_hbm,
                  x_v, src_v, dst_v, vals_v, stage_v, agg_sh):
        cid = lax.axis_index("c")
        sid = lax.axis_index("s")
        wid = sid * 2 + cid

        pltpu.sync_copy(z_hbm.at[pl.ds(sid * PER_TILE, PER_TILE)], stage_v)
        pltpu.sync_copy(stage_v, agg_sh.at[pl.ds(sid * PER_TILE, PER_TILE)])
        pltpu.sync_copy(x_hbm, x_v)
        plsc.subcore_barrier()

        def chunk_body(k, carry):
            chunk = wid + N_WORKERS * k

            @pl.when(chunk < N_CHUNKS)
            def _():
                row0 = chunk * CHUNK_ROWS
                pltpu.sync_copy(src_hbm.at[pl.ds(row0, CHUNK_ROWS)], src_v)
                pltpu.sync_copy(dst_hbm.at[pl.ds(row0, CHUNK_ROWS)], dst_v)
                for r in range(CHUNK_ROWS):
                    for t in range(LANES // 16):
                        idx = src_v[r, pl.ds(t * 16, 16)]
                        vals_v[r, pl.ds(t * 16, 16)] = plsc.load_gather(
                            x_v, [idx])
                for r in range(CHUNK_ROWS):
                    pltpu.sync_copy(vals_v.at[r], agg_sh.at[dst_v.at[r]],
                                    add=True)

            return carry

        lax.fori_loop(0, MAX_CHUNKS_PER_TILE, chunk_body, 0)
        plsc.subcore_barrier()

        pltpu.sync_copy(agg_sh.at[pl.ds(sid * PER_TILE, PER_TILE)], stage_v)

        @pl.when(cid == 0)
        def _():
            pltpu.sync_copy(stage_v, p0_hbm.at[pl.ds(sid * PER_TILE, PER_TILE)])

        @pl.when(cid == 1)
        def _():
            pltpu.sync_copy(stage_v, p1_hbm.at[pl.ds(sid * PER_TILE, PER_TILE)])

    return sc_kernel(x_pad, src2d, dst2d, zeros_pad)


def _mlp_body(eps_r, w1_r, b1_r, w2_r, b2_r, w3_r, b3_r,
              x_r, p0_r, p1_r, o_r):
    s = (1.0 + eps_r[0]) * x_r[...] + p0_r[...] + p1_r[...]
    h1 = [jnp.maximum(s * w1_r[j] + b1_r[j], 0.0) for j in range(20)]
    acc = None
    for k in range(20):
        hk = h1[0] * w2_r[k, 0]
        for j in range(1, 20):
            hk = hk + h1[j] * w2_r[k, j]
        hk = jnp.maximum(hk + b2_r[k], 0.0)
        acc = hk * w3_r[k] if acc is None else acc + hk * w3_r[k]
    o_r[...] = acc + b3_r[0]


def _mlp(x2d, p02d, p12d, W1, b1, W2, b2, W3, b3, eps):
    smem = pl.BlockSpec(memory_space=pltpu.SMEM)
    vmem = pl.BlockSpec(memory_space=pltpu.VMEM)
    return pl.pallas_call(
        _mlp_body,
        out_shape=jax.ShapeDtypeStruct(x2d.shape, jnp.float32),
        in_specs=[smem] * 7 + [vmem] * 3,
        out_specs=vmem,
    )(jnp.reshape(eps, (1,)), jnp.reshape(W1, (20,)), b1, W2, b2,
      jnp.reshape(W3, (20,)), b3, x2d, p02d, p12d)


def kernel(x, edge_index, W1, b1, W2, b2, W3, b3, eps):
    x_flat = jnp.reshape(x, (N_NODES,))
    x_pad = jnp.pad(x_flat, (0, N_PAD - N_NODES))
    src2d = jnp.reshape(edge_index[0], (ROWS, LANES))
    dst2d = jnp.reshape(edge_index[1], (ROWS, LANES))
    zeros_pad = jnp.zeros((N_PAD,), jnp.float32)

    p0, p1 = _sc_scatter(x_pad, src2d, dst2d, zeros_pad)

    out2d = _mlp(
        jnp.reshape(x_pad, (N_PAD // LANES, LANES)),
        jnp.reshape(p0, (N_PAD // LANES, LANES)),
        jnp.reshape(p1, (N_PAD // LANES, LANES)),
        W1, b1, W2, b2, W3, b3, eps)
    return jnp.reshape(jnp.reshape(out2d, (N_PAD,))[:N_NODES], (N_NODES, 1))


# speedup vs baseline: 182.6783x; 1.1793x over previous
import functools

import jax
import jax.numpy as jnp
from jax import lax
from jax.experimental import pallas as pl
from jax.experimental.pallas import tpu as pltpu
from jax.experimental.pallas import tpu_sc as plsc

N_NODES = 100000
N_EDGES = 6400000
LANES = 128
N_PAD = 100352
PER_TILE = N_PAD // 16
N_WORKERS = 32
CHUNK_ROWS = 16
CHUNK_EDGES = CHUNK_ROWS * LANES
K_CHUNKS = 99
TOT_EDGES = (N_WORKERS * K_CHUNKS + 1) * CHUNK_EDGES
TOT_ROWS = TOT_EDGES // LANES
NBUF = 3


def _sc_scatter(x_pad, src2d, dst2d, zeros_pad):
    mesh = plsc.VectorSubcoreMesh(core_axis_name="c", subcore_axis_name="s")

    @functools.partial(
        pl.kernel,
        out_type=[
            jax.ShapeDtypeStruct((N_PAD,), jnp.float32),
            jax.ShapeDtypeStruct((N_PAD,), jnp.float32),
        ],
        mesh=mesh,
        compiler_params=pltpu.CompilerParams(needs_layout_passes=False),
        scratch_types=[
            pltpu.VMEM((N_PAD,), jnp.float32),
            pltpu.VMEM((NBUF * CHUNK_ROWS, LANES), jnp.int32),
            pltpu.VMEM((NBUF * CHUNK_ROWS, LANES), jnp.int32),
            pltpu.VMEM((NBUF * CHUNK_ROWS, LANES), jnp.float32),
            pltpu.VMEM_SHARED((N_PAD,), jnp.float32),
        ] + [pltpu.SemaphoreType.DMA] * (2 * NBUF + CHUNK_ROWS),
    )
    def sc_kernel(x_hbm, src_hbm, dst_hbm, z_hbm, p0_hbm, p1_hbm,
                  x_v, src_v, dst_v, vals_v, agg_sh, *sems):
        sem_si = sems[0:NBUF]
        sem_di = sems[NBUF:2 * NBUF]
        sem_sc = sems[2 * NBUF:]
        cid = lax.axis_index("c")
        sid = lax.axis_index("s")
        wid = sid * 2 + cid

        pltpu.sync_copy(z_hbm.at[pl.ds(sid * PER_TILE, PER_TILE)],
                        agg_sh.at[pl.ds(sid * PER_TILE, PER_TILE)])
        pltpu.sync_copy(x_hbm, x_v)
        plsc.subcore_barrier()

        row_base = wid * K_CHUNKS * CHUNK_ROWS

        def issue_inputs(k, bb):
            row0 = row_base + k * CHUNK_ROWS
            pltpu.async_copy(src_hbm.at[pl.ds(row0, CHUNK_ROWS)],
                             src_v.at[pl.ds(bb * CHUNK_ROWS, CHUNK_ROWS)],
                             sem_si[bb])
            pltpu.async_copy(dst_hbm.at[pl.ds(row0, CHUNK_ROWS)],
                             dst_v.at[pl.ds(bb * CHUNK_ROWS, CHUNK_ROWS)],
                             sem_di[bb])

        def wait_inputs(bb):
            pltpu.make_async_copy(src_hbm.at[pl.ds(0, CHUNK_ROWS)],
                                  src_v.at[pl.ds(bb * CHUNK_ROWS, CHUNK_ROWS)],
                                  sem_si[bb]).wait()
            pltpu.make_async_copy(dst_hbm.at[pl.ds(0, CHUNK_ROWS)],
                                  dst_v.at[pl.ds(bb * CHUNK_ROWS, CHUNK_ROWS)],
                                  sem_di[bb]).wait()

        def gather(bb):
            for r in range(CHUNK_ROWS):
                for t in range(LANES // 16):
                    idx = src_v[bb * CHUNK_ROWS + r, pl.ds(t * 16, 16)]
                    vals_v[bb * CHUNK_ROWS + r, pl.ds(t * 16, 16)] = (
                        plsc.load_gather(x_v, [idx]))

        def step(k, bb, first_round):
            issue_inputs(k + 1, (bb + 1) % NBUF)
            wait_inputs(bb)
            gather(bb)
            for r in range(CHUNK_ROWS):
                pltpu.sync_copy(vals_v.at[bb * CHUNK_ROWS + r],
                                agg_sh.at[dst_v.at[bb * CHUNK_ROWS + r]],
                                add=True)

        issue_inputs(0, 0)
        for k0 in range(NBUF):
            step(k0, k0, k0 < NBUF - 1)

        def body(ko, carry):
            for b in range(NBUF):
                step(ko * NBUF + b, b, False)
            return carry

        lax.fori_loop(1, K_CHUNKS // NBUF, body, 0)
        wait_inputs(K_CHUNKS % NBUF)
        plsc.subcore_barrier()

        @pl.when(cid == 0)
        def _():
            pltpu.sync_copy(agg_sh.at[pl.ds(sid * PER_TILE, PER_TILE)],
                            p0_hbm.at[pl.ds(sid * PER_TILE, PER_TILE)])

        @pl.when(cid == 1)
        def _():
            pltpu.sync_copy(agg_sh.at[pl.ds(sid * PER_TILE, PER_TILE)],
                            p1_hbm.at[pl.ds(sid * PER_TILE, PER_TILE)])

    return sc_kernel(x_pad, src2d, dst2d, zeros_pad)


def _mlp_body(eps_r, w1_r, b1_r, w2_r, b2_r, w3_r, b3_r,
              x_r, p0_r, p1_r, o_r):
    s = (1.0 + eps_r[0]) * x_r[...] + p0_r[...] + p1_r[...]
    h1 = [jnp.maximum(s * w1_r[j] + b1_r[j], 0.0) for j in range(20)]
    acc = None
    for k in range(20):
        hk = h1[0] * w2_r[k, 0]
        for j in range(1, 20):
            hk = hk + h1[j] * w2_r[k, j]
        hk = jnp.maximum(hk + b2_r[k], 0.0)
        acc = hk * w3_r[k] if acc is None else acc + hk * w3_r[k]
    o_r[...] = acc + b3_r[0]


def _mlp(x2d, p02d, p12d, W1, b1, W2, b2, W3, b3, eps):
    smem = pl.BlockSpec(memory_space=pltpu.SMEM)
    vmem = pl.BlockSpec(memory_space=pltpu.VMEM)
    return pl.pallas_call(
        _mlp_body,
        out_shape=jax.ShapeDtypeStruct(x2d.shape, jnp.float32),
        in_specs=[smem] * 7 + [vmem] * 3,
        out_specs=vmem,
    )(jnp.reshape(eps, (1,)), jnp.reshape(W1, (20,)), b1, W2, b2,
      jnp.reshape(W3, (20,)), b3, x2d, p02d, p12d)


def kernel(x, edge_index, W1, b1, W2, b2, W3, b3, eps):
    x_flat = jnp.reshape(x, (N_NODES,))
    x_pad = jnp.pad(x_flat, (0, N_PAD - N_NODES))
    pad_e = TOT_EDGES - N_EDGES
    src2d = jnp.reshape(
        jnp.concatenate([edge_index[0],
                         jnp.zeros((pad_e,), jnp.int32)]),
        (TOT_ROWS, LANES))
    dst2d = jnp.reshape(
        jnp.concatenate([edge_index[1],
                         jnp.full((pad_e,), N_PAD - 1, jnp.int32)]),
        (TOT_ROWS, LANES))
    zeros_pad = jnp.zeros((N_PAD,), jnp.float32)

    p0, p1 = _sc_scatter(x_pad, src2d, dst2d, zeros_pad)

    out2d = _mlp(
        jnp.reshape(x_pad, (N_PAD // LANES, LANES)),
        jnp.reshape(p0, (N_PAD // LANES, LANES)),
        jnp.reshape(p1, (N_PAD // LANES, LANES)),
        W1, b1, W2, b2, W3, b3, eps)
    return jnp.reshape(jnp.reshape(out2d, (N_PAD,))[:N_NODES], (N_NODES, 1))


# speedup vs baseline: 250.5689x; 1.3716x over previous
import functools

import jax
import jax.numpy as jnp
from jax import lax
from jax.experimental import pallas as pl
from jax.experimental.pallas import tpu as pltpu
from jax.experimental.pallas import tpu_sc as plsc

N_NODES = 100000
N_EDGES = 6400000
LANES = 128
N_PAD = 100352
PER_TILE = N_PAD // 16
N_WORKERS = 32
CHUNK_ROWS = 16
CHUNK_EDGES = CHUNK_ROWS * LANES
TOT_ROWS = N_EDGES // LANES
N_CHUNKS = TOT_ROWS // CHUNK_ROWS
K_MAIN = N_CHUNKS // N_WORKERS
N_TAIL = N_CHUNKS - K_MAIN * N_WORKERS
NBUF = 3


def _sc_scatter(x_pad, src2d, dst2d, zeros_pad):
    mesh = plsc.VectorSubcoreMesh(core_axis_name="c", subcore_axis_name="s")

    @functools.partial(
        pl.kernel,
        out_type=[
            jax.ShapeDtypeStruct((N_PAD,), jnp.float32),
            jax.ShapeDtypeStruct((N_PAD,), jnp.float32),
        ],
        mesh=mesh,
        compiler_params=pltpu.CompilerParams(needs_layout_passes=False),
        scratch_types=[
            pltpu.VMEM((N_PAD,), jnp.float32),
            pltpu.VMEM((NBUF * CHUNK_ROWS, LANES), jnp.int32),
            pltpu.VMEM((NBUF * CHUNK_ROWS, LANES), jnp.int32),
            pltpu.VMEM((NBUF * CHUNK_ROWS, LANES), jnp.float32),
            pltpu.VMEM_SHARED((N_PAD,), jnp.float32),
        ] + [pltpu.SemaphoreType.DMA] * (2 * NBUF + CHUNK_ROWS),
    )
    def sc_kernel(x_hbm, src_hbm, dst_hbm, z_hbm, p0_hbm, p1_hbm,
                  x_v, src_v, dst_v, vals_v, agg_sh, *sems):
        sem_si = sems[0:NBUF]
        sem_di = sems[NBUF:2 * NBUF]
        sem_sc = sems[2 * NBUF:]
        cid = lax.axis_index("c")
        sid = lax.axis_index("s")
        wid = sid * 2 + cid

        pltpu.sync_copy(z_hbm.at[pl.ds(sid * PER_TILE, PER_TILE)],
                        agg_sh.at[pl.ds(sid * PER_TILE, PER_TILE)])
        pltpu.sync_copy(x_hbm, x_v)
        plsc.subcore_barrier()

        row_base = wid * K_MAIN * CHUNK_ROWS

        def issue_inputs(k, bb):
            row0 = row_base + k * CHUNK_ROWS
            pltpu.async_copy(src_hbm.at[pl.ds(row0, CHUNK_ROWS)],
                             src_v.at[pl.ds(bb * CHUNK_ROWS, CHUNK_ROWS)],
                             sem_si[bb])
            pltpu.async_copy(dst_hbm.at[pl.ds(row0, CHUNK_ROWS)],
                             dst_v.at[pl.ds(bb * CHUNK_ROWS, CHUNK_ROWS)],
                             sem_di[bb])

        def wait_inputs(bb):
            pltpu.make_async_copy(src_hbm.at[pl.ds(0, CHUNK_ROWS)],
                                  src_v.at[pl.ds(bb * CHUNK_ROWS, CHUNK_ROWS)],
                                  sem_si[bb]).wait()
            pltpu.make_async_copy(dst_hbm.at[pl.ds(0, CHUNK_ROWS)],
                                  dst_v.at[pl.ds(bb * CHUNK_ROWS, CHUNK_ROWS)],
                                  sem_di[bb]).wait()

        def gather_row(bb, r):
            for t in range(LANES // 16):
                idx = src_v[bb * CHUNK_ROWS + r, pl.ds(t * 16, 16)]
                vals_v[bb * CHUNK_ROWS + r, pl.ds(t * 16, 16)] = (
                    plsc.load_gather(x_v, [idx]))

        def fire_row(bb, r):
            return pltpu.async_copy(
                vals_v.at[bb * CHUNK_ROWS + r],
                agg_sh.at[dst_v.at[bb * CHUNK_ROWS + r]], sem_sc[0],
                add=True)

        def step(k, bb):
            issue_inputs(k + 1, (bb + 1) % NBUF)
            wait_inputs(bb)
            gather_row(bb, 0)
            for r in range(CHUNK_ROWS):
                d = fire_row(bb, r)
                if r + 1 < CHUNK_ROWS:
                    gather_row(bb, r + 1)
                d.wait()

        issue_inputs(0, 0)
        for k0 in range(NBUF):
            step(k0, k0)

        def body(ko, carry):
            for b in range(NBUF):
                step(ko * NBUF + b, b)
            return carry

        lax.fori_loop(1, (K_MAIN - 1) // NBUF, body, 0)
        step(K_MAIN - 1, (K_MAIN - 1) % NBUF)
        wait_inputs(K_MAIN % NBUF)

        @pl.when(wid < N_TAIL)
        def _():
            row0 = (K_MAIN * N_WORKERS + wid) * CHUNK_ROWS
            pltpu.sync_copy(src_hbm.at[pl.ds(row0, CHUNK_ROWS)],
                            src_v.at[pl.ds(0, CHUNK_ROWS)])
            pltpu.sync_copy(dst_hbm.at[pl.ds(row0, CHUNK_ROWS)],
                            dst_v.at[pl.ds(0, CHUNK_ROWS)])
            gather_row(0, 0)
            for r in range(CHUNK_ROWS):
                d = fire_row(0, r)
                if r + 1 < CHUNK_ROWS:
                    gather_row(0, r + 1)
                d.wait()

        plsc.subcore_barrier()

        @pl.when(cid == 0)
        def _():
            pltpu.sync_copy(agg_sh.at[pl.ds(sid * PER_TILE, PER_TILE)],
                            p0_hbm.at[pl.ds(sid * PER_TILE, PER_TILE)])

        @pl.when(cid == 1)
        def _():
            pltpu.sync_copy(agg_sh.at[pl.ds(sid * PER_TILE, PER_TILE)],
                            p1_hbm.at[pl.ds(sid * PER_TILE, PER_TILE)])

    return sc_kernel(x_pad, src2d, dst2d, zeros_pad)


def _mlp_body(eps_r, w1_r, b1_r, w2_r, b2_r, w3_r, b3_r,
              x_r, p0_r, p1_r, o_r):
    s = (1.0 + eps_r[0]) * x_r[...] + p0_r[...] + p1_r[...]
    h1 = [jnp.maximum(s * w1_r[j] + b1_r[j], 0.0) for j in range(20)]
    acc = None
    for k in range(20):
        hk = h1[0] * w2_r[k, 0]
        for j in range(1, 20):
            hk = hk + h1[j] * w2_r[k, j]
        hk = jnp.maximum(hk + b2_r[k], 0.0)
        acc = hk * w3_r[k] if acc is None else acc + hk * w3_r[k]
    o_r[...] = acc + b3_r[0]


def _mlp(x2d, p02d, p12d, W1, b1, W2, b2, W3, b3, eps):
    smem = pl.BlockSpec(memory_space=pltpu.SMEM)
    vmem = pl.BlockSpec(memory_space=pltpu.VMEM)
    return pl.pallas_call(
        _mlp_body,
        out_shape=jax.ShapeDtypeStruct(x2d.shape, jnp.float32),
        in_specs=[smem] * 7 + [vmem] * 3,
        out_specs=vmem,
    )(jnp.reshape(eps, (1,)), jnp.reshape(W1, (20,)), b1, W2, b2,
      jnp.reshape(W3, (20,)), b3, x2d, p02d, p12d)


def kernel(x, edge_index, W1, b1, W2, b2, W3, b3, eps):
    x_flat = jnp.reshape(x, (N_NODES,))
    x_pad = jnp.pad(x_flat, (0, N_PAD - N_NODES))
    src2d = jnp.reshape(edge_index[0], (TOT_ROWS, LANES))
    dst2d = jnp.reshape(edge_index[1], (TOT_ROWS, LANES))
    zeros_pad = jnp.zeros((N_PAD,), jnp.float32)

    p0, p1 = _sc_scatter(x_pad, src2d, dst2d, zeros_pad)

    out2d = _mlp(
        jnp.reshape(x_pad, (N_PAD // LANES, LANES)),
        jnp.reshape(p0, (N_PAD // LANES, LANES)),
        jnp.reshape(p1, (N_PAD // LANES, LANES)),
        W1, b1, W2, b2, W3, b3, eps)
    return jnp.reshape(jnp.reshape(out2d, (N_PAD,))[:N_NODES], (N_NODES, 1))
